# Initial kernel scaffold; baseline (speedup 1.0000x reference)
import jax
import jax.numpy as jnp
from jax.experimental import pallas as pl


def kernel(features, edge_index, W0, b0, W1, b1, W2, b2):
    raise NotImplementedError("write your pallas kernel here")


# speedup vs baseline: 7.7697x; 7.7697x over previous
import functools

import jax
import jax.numpy as jnp
from jax import lax
from jax.experimental import pallas as pl
from jax.experimental.pallas import tpu as pltpu
from jax.experimental.pallas import tpu_sc as plsc

N = 10000
E = 320000
D = 128
H = 256
C = 64
K_PROP = 10
ALPHA = 0.1

NC = 2
NS = 16
NW = NC * NS
EPW = E // NW
CE = 80
NB = EPW // CE
RPT = N // NS

_mesh = plsc.VectorSubcoreMesh(core_axis_name="c", subcore_axis_name="s",
                               num_cores=NC, num_subcores=NS)
_sc_params = pltpu.CompilerParams(use_tc_tiling_on_sc=False)


def _worker_id():
    return lax.axis_index("s") * NC + lax.axis_index("c")


@functools.partial(
    pl.kernel,
    out_type=jax.ShapeDtypeStruct((NC, 2, N), jnp.float32),
    mesh=_mesh,
    compiler_params=_sc_params,
    scratch_types=[
        pltpu.VMEM((NB, CE), jnp.int32),
        pltpu.VMEM((NB, CE), jnp.int32),
        pltpu.VMEM((CE,), jnp.float32),
        pltpu.VMEM((2000,), jnp.float32),
        pltpu.VMEM_SHARED((N,), jnp.float32),
        pltpu.VMEM_SHARED((N,), jnp.float32),
    ],
)
def _deg_kernel(src_hbm, dst_hbm, dpart, src_v, dst_v, ones_v, z_v,
                acc_out, acc_in):
    cid = lax.axis_index("c")
    sid = lax.axis_index("s")
    wid = _worker_id()

    def fill_ones(i, _):
        ones_v[pl.ds(i * 16, 16)] = jnp.ones((16,), jnp.float32)
        return 0
    lax.fori_loop(0, CE // 16, fill_ones, 0)

    def fill_z(i, _):
        z_v[pl.ds(i * 16, 16)] = jnp.zeros((16,), jnp.float32)
        return 0
    lax.fori_loop(0, 2000 // 16, fill_z, 0)

    @pl.when(sid < 5)
    def _():
        pltpu.sync_copy(z_v, acc_out.at[pl.ds(sid * 2000, 2000)])
        pltpu.sync_copy(z_v, acc_in.at[pl.ds(sid * 2000, 2000)])

    plsc.subcore_barrier()

    pltpu.sync_copy(src_hbm.at[wid], src_v)
    pltpu.sync_copy(dst_hbm.at[wid], dst_v)

    def body(j, _):
        pltpu.sync_copy(ones_v, acc_out.at[src_v.at[j]], add=True)
        pltpu.sync_copy(ones_v, acc_in.at[dst_v.at[j]], add=True)
        return 0
    lax.fori_loop(0, NB, body, 0)

    plsc.subcore_barrier()

    @pl.when(sid < 10)
    def _():
        pltpu.sync_copy(acc_out.at[pl.ds(sid * 1000, 1000)],
                        dpart.at[cid, 0, pl.ds(sid * 1000, 1000)])
        pltpu.sync_copy(acc_in.at[pl.ds(sid * 1000, 1000)],
                        dpart.at[cid, 1, pl.ds(sid * 1000, 1000)])


_BR = 1000


def _mlp_body(feat, w0, b0, w1, b1, w2, b2, dpo, dpi,
              s0_ref, h0a_ref, nin_ref, nout_ref):
    x = feat[...]
    h = jnp.maximum(jnp.dot(x, w0[...], preferred_element_type=jnp.float32)
                    + b0[...], 0.0)
    h = jnp.maximum(jnp.dot(h, w1[...], preferred_element_type=jnp.float32)
                    + b1[...], 0.0)
    h = jnp.dot(h, w2[...], preferred_element_type=jnp.float32) + b2[...]

    dout = jnp.maximum(dpo[0] + dpo[1], 1.0)
    din = jnp.maximum(dpi[0] + dpi[1], 1.0)
    no = lax.rsqrt(dout)
    ni = lax.rsqrt(din)

    s0_ref[...] = h * no
    h0a_ref[...] = ALPHA * h
    nin_ref[...] = jnp.broadcast_to((1.0 - ALPHA) * ni, (_BR, C))
    nout_ref[...] = jnp.broadcast_to(no, (_BR, C))


def _mlp_kernel(features, W0, b0, W1, b1, W2, b2, dpo, dpi):
    grid = (N // _BR,)
    out4 = jax.ShapeDtypeStruct((N, C), jnp.float32)
    return pl.pallas_call(
        _mlp_body,
        grid=grid,
        in_specs=[
            pl.BlockSpec((_BR, D), lambda i: (i, 0)),
            pl.BlockSpec((D, H), lambda i: (0, 0)),
            pl.BlockSpec((1, H), lambda i: (0, 0)),
            pl.BlockSpec((H, H), lambda i: (0, 0)),
            pl.BlockSpec((1, H), lambda i: (0, 0)),
            pl.BlockSpec((H, C), lambda i: (0, 0)),
            pl.BlockSpec((1, C), lambda i: (0, 0)),
            pl.BlockSpec((NC, _BR, 1), lambda i: (0, i, 0)),
            pl.BlockSpec((NC, _BR, 1), lambda i: (0, i, 0)),
        ],
        out_specs=[pl.BlockSpec((_BR, C), lambda i: (i, 0))] * 4,
        out_shape=[out4, out4, out4, out4],
    )(features, W0, b0.reshape(1, H), W1, b1.reshape(1, H),
      W2, b2.reshape(1, C), dpo, dpi)


@functools.partial(
    pl.kernel,
    out_type=jax.ShapeDtypeStruct((NC, N, C), jnp.float32),
    mesh=_mesh,
    compiler_params=_sc_params,
    scratch_types=[
        pltpu.VMEM((NB, CE), jnp.int32),
        pltpu.VMEM((NB, CE), jnp.int32),
        pltpu.VMEM((CE, C), jnp.float32),
        pltpu.VMEM((25, C), jnp.float32),
        pltpu.VMEM_SHARED((N, C), jnp.float32),
        pltpu.SemaphoreType.DMA,
    ],
)
def _step_kernel(s_hbm, src_hbm, dst_hbm, part, src_v, dst_v, rows_v, z_v,
                 acc, sem):
    cid = lax.axis_index("c")
    sid = lax.axis_index("s")
    wid = _worker_id()

    def fill_z(i, _):
        r = i // (C // 16)
        col = (i % (C // 16)) * 16
        z_v[r, pl.ds(col, 16)] = jnp.zeros((16,), jnp.float32)
        return 0
    lax.fori_loop(0, 25 * (C // 16), fill_z, 0)

    def zero_acc(t, _):
        pltpu.sync_copy(z_v, acc.at[pl.ds(sid * RPT + t * 25, 25)])
        return 0
    lax.fori_loop(0, RPT // 25, zero_acc, 0)

    pltpu.sync_copy(src_hbm.at[wid], src_v)
    pltpu.sync_copy(dst_hbm.at[wid], dst_v)

    plsc.subcore_barrier()

    def body(j, _):
        pltpu.async_copy(s_hbm.at[src_v.at[j]], rows_v, sem).wait()
        pltpu.sync_copy(rows_v, acc.at[dst_v.at[j]], add=True)
        return 0
    lax.fori_loop(0, NB, body, 0)

    plsc.subcore_barrier()

    pltpu.sync_copy(acc.at[pl.ds(sid * RPT, RPT)],
                    part.at[cid, pl.ds(sid * RPT, RPT)])


_FLAT = N * C
_FPW = _FLAT // NW
_CH = 10000


def _combine_body(scale_out, p_hbm, nin_hbm, h0a_hbm, nout_hbm, out_hbm,
                  p0_v, p1_v, nin_v, h0a_v, nout_v, o_v):
    wid = _worker_id()
    base = wid * _FPW

    def chunk(cnk, _):
        off = base + cnk * _CH
        pltpu.sync_copy(p_hbm.at[0, pl.ds(off, _CH)], p0_v)
        pltpu.sync_copy(p_hbm.at[1, pl.ds(off, _CH)], p1_v)
        pltpu.sync_copy(nin_hbm.at[pl.ds(off, _CH)], nin_v)
        pltpu.sync_copy(h0a_hbm.at[pl.ds(off, _CH)], h0a_v)
        if scale_out:
            pltpu.sync_copy(nout_hbm.at[pl.ds(off, _CH)], nout_v)

        def body(i, _):
            sl = pl.ds(i * 16, 16)
            v = (p0_v[sl] + p1_v[sl]) * nin_v[sl] + h0a_v[sl]
            if scale_out:
                v = v * nout_v[sl]
            o_v[sl] = v
            return 0
        lax.fori_loop(0, _CH // 16, body, 0)
        pltpu.sync_copy(o_v, out_hbm.at[pl.ds(off, _CH)])
        return 0
    lax.fori_loop(0, _FPW // _CH, chunk, 0)


def _make_combine(scale_out):
    return pl.kernel(
        functools.partial(_combine_body, scale_out),
        out_type=jax.ShapeDtypeStruct((_FLAT,), jnp.float32),
        mesh=_mesh,
        compiler_params=_sc_params,
        scratch_types=[
            pltpu.VMEM((_CH,), jnp.float32),
            pltpu.VMEM((_CH,), jnp.float32),
            pltpu.VMEM((_CH,), jnp.float32),
            pltpu.VMEM((_CH,), jnp.float32),
            pltpu.VMEM((_CH,), jnp.float32),
            pltpu.VMEM((_CH,), jnp.float32),
        ],
    )


_combine_mid = _make_combine(True)
_combine_last = _make_combine(False)


def kernel(features, edge_index, W0, b0, W1, b1, W2, b2):
    src = edge_index[0].reshape(NW, NB, CE)
    dst = edge_index[1].reshape(NW, NB, CE)

    dpart = _deg_kernel(src, dst)
    dpo = dpart[:, 0, :].reshape(NC, N, 1)
    dpi = dpart[:, 1, :].reshape(NC, N, 1)

    s0, h0a, nin1, nout = _mlp_kernel(features, W0, b0, W1, b1, W2, b2,
                                      dpo, dpi)

    h0a_f = h0a.reshape(_FLAT)
    nin1_f = nin1.reshape(_FLAT)
    nout_f = nout.reshape(_FLAT)

    s = s0
    for k in range(K_PROP):
        part = _step_kernel(s, src, dst)
        p_f = part.reshape(NC, _FLAT)
        if k < K_PROP - 1:
            s = _combine_mid(p_f, nin1_f, h0a_f, nout_f).reshape(N, C)
        else:
            s = _combine_last(p_f, nin1_f, h0a_f, nout_f).reshape(N, C)
    return s


# speedup vs baseline: 13.9183x; 1.7913x over previous
import functools

import jax
import jax.numpy as jnp
from jax import lax
from jax.experimental import pallas as pl
from jax.experimental.pallas import tpu as pltpu
from jax.experimental.pallas import tpu_sc as plsc

N = 10000
E = 320000
D = 128
H = 256
C = 64
K_PROP = 10
ALPHA = 0.1

NC = 2
NS = 16
NW = NC * NS
EPW = E // NW
CE = 125
NB = EPW // CE
RPT = N // NS

_mesh = plsc.VectorSubcoreMesh(core_axis_name="c", subcore_axis_name="s",
                               num_cores=NC, num_subcores=NS)
_sc_params = pltpu.CompilerParams(use_tc_tiling_on_sc=False)


def _worker_id():
    return lax.axis_index("s") * NC + lax.axis_index("c")


@functools.partial(
    pl.kernel,
    out_type=jax.ShapeDtypeStruct((NC, 2, N), jnp.float32),
    mesh=_mesh,
    compiler_params=_sc_params,
    scratch_types=[
        pltpu.VMEM((NB, CE), jnp.int32),
        pltpu.VMEM((NB, CE), jnp.int32),
        pltpu.VMEM((128,), jnp.float32),
        pltpu.VMEM((2000,), jnp.float32),
        pltpu.VMEM_SHARED((N,), jnp.float32),
        pltpu.VMEM_SHARED((N,), jnp.float32),
    ],
)
def _deg_kernel(src_hbm, dst_hbm, dpart, src_v, dst_v, ones_v, z_v,
                acc_out, acc_in):
    cid = lax.axis_index("c")
    sid = lax.axis_index("s")
    wid = _worker_id()

    def fill_ones(i, _):
        ones_v[pl.ds(i * 16, 16)] = jnp.ones((16,), jnp.float32)
        return 0
    lax.fori_loop(0, 128 // 16, fill_ones, 0)

    def fill_z(i, _):
        z_v[pl.ds(i * 16, 16)] = jnp.zeros((16,), jnp.float32)
        return 0
    lax.fori_loop(0, 2000 // 16, fill_z, 0)

    @pl.when(sid < 5)
    def _():
        pltpu.sync_copy(z_v, acc_out.at[pl.ds(sid * 2000, 2000)])
        pltpu.sync_copy(z_v, acc_in.at[pl.ds(sid * 2000, 2000)])

    plsc.subcore_barrier()

    pltpu.sync_copy(src_hbm.at[wid], src_v)
    pltpu.sync_copy(dst_hbm.at[wid], dst_v)

    def body(j, _):
        pltpu.sync_copy(ones_v.at[pl.ds(0, CE)], acc_out.at[src_v.at[j]],
                        add=True)
        pltpu.sync_copy(ones_v.at[pl.ds(0, CE)], acc_in.at[dst_v.at[j]],
                        add=True)
        return 0
    lax.fori_loop(0, NB, body, 0)

    plsc.subcore_barrier()

    @pl.when(sid < 10)
    def _():
        pltpu.sync_copy(acc_out.at[pl.ds(sid * 1000, 1000)],
                        dpart.at[cid, 0, pl.ds(sid * 1000, 1000)])
        pltpu.sync_copy(acc_in.at[pl.ds(sid * 1000, 1000)],
                        dpart.at[cid, 1, pl.ds(sid * 1000, 1000)])


_BR = 1000


def _mlp_body(feat, w0, b0, w1, b1, w2, b2, dpo, dpi,
              s0_ref, h0a_ref, nin_ref, nout_ref):
    x = feat[...]
    h = jnp.maximum(jnp.dot(x, w0[...], preferred_element_type=jnp.float32)
                    + b0[...], 0.0)
    h = jnp.maximum(jnp.dot(h, w1[...], preferred_element_type=jnp.float32)
                    + b1[...], 0.0)
    h = jnp.dot(h, w2[...], preferred_element_type=jnp.float32) + b2[...]

    dout = jnp.maximum(dpo[0] + dpo[1], 1.0)
    din = jnp.maximum(dpi[0] + dpi[1], 1.0)
    no = lax.rsqrt(dout)
    ni = lax.rsqrt(din)

    s0_ref[...] = h * no
    h0a_ref[...] = ALPHA * h
    nin_ref[...] = jnp.broadcast_to((1.0 - ALPHA) * ni, (_BR, C))
    nout_ref[...] = jnp.broadcast_to(no, (_BR, C))


def _mlp_kernel(features, W0, b0, W1, b1, W2, b2, dpo, dpi):
    grid = (N // _BR,)
    out4 = jax.ShapeDtypeStruct((N, C), jnp.float32)
    return pl.pallas_call(
        _mlp_body,
        grid=grid,
        in_specs=[
            pl.BlockSpec((_BR, D), lambda i: (i, 0)),
            pl.BlockSpec((D, H), lambda i: (0, 0)),
            pl.BlockSpec((1, H), lambda i: (0, 0)),
            pl.BlockSpec((H, H), lambda i: (0, 0)),
            pl.BlockSpec((1, H), lambda i: (0, 0)),
            pl.BlockSpec((H, C), lambda i: (0, 0)),
            pl.BlockSpec((1, C), lambda i: (0, 0)),
            pl.BlockSpec((NC, _BR, 1), lambda i: (0, i, 0)),
            pl.BlockSpec((NC, _BR, 1), lambda i: (0, i, 0)),
        ],
        out_specs=[pl.BlockSpec((_BR, C), lambda i: (i, 0))] * 4,
        out_shape=[out4, out4, out4, out4],
    )(features, W0, b0.reshape(1, H), W1, b1.reshape(1, H),
      W2, b2.reshape(1, C), dpo, dpi)


@functools.partial(
    pl.kernel,
    out_type=jax.ShapeDtypeStruct((NC, N, C), jnp.float32),
    mesh=_mesh,
    compiler_params=_sc_params,
    scratch_types=[
        pltpu.VMEM((NB, CE), jnp.int32),
        pltpu.VMEM((NB, CE), jnp.int32),
        pltpu.VMEM((2, CE, C), jnp.float32),
        pltpu.VMEM((25, C), jnp.float32),
        pltpu.VMEM_SHARED((N, C), jnp.float32),
        pltpu.SemaphoreType.DMA((2,)),
        pltpu.SemaphoreType.DMA((2,)),
    ],
)
def _step_kernel(s_hbm, src_hbm, dst_hbm, part, src_v, dst_v, rows_v, z_v,
                 acc, gsem, ssem):
    cid = lax.axis_index("c")
    sid = lax.axis_index("s")
    wid = _worker_id()

    def fill_z(i, _):
        r = i // (C // 16)
        col = (i % (C // 16)) * 16
        z_v[r, pl.ds(col, 16)] = jnp.zeros((16,), jnp.float32)
        return 0
    lax.fori_loop(0, 25 * (C // 16), fill_z, 0)

    def zero_acc(t, _):
        pltpu.sync_copy(z_v, acc.at[pl.ds(sid * RPT + t * 25, 25)])
        return 0
    lax.fori_loop(0, RPT // 25, zero_acc, 0)

    pltpu.sync_copy(src_hbm.at[wid], src_v)
    pltpu.sync_copy(dst_hbm.at[wid], dst_v)

    plsc.subcore_barrier()

    pltpu.async_copy(s_hbm.at[src_v.at[0]], rows_v.at[0], gsem.at[0])

    def body(j, _):
        b = j % 2
        nb = (j + 1) % 2

        @pl.when(j >= 1)
        def _():
            pltpu.make_async_copy(rows_v.at[nb], acc.at[dst_v.at[j - 1]],
                                  ssem.at[nb]).wait()

        @pl.when(j + 1 < NB)
        def _():
            pltpu.async_copy(s_hbm.at[src_v.at[j + 1]], rows_v.at[nb],
                             gsem.at[nb])

        pltpu.make_async_copy(s_hbm.at[src_v.at[j]], rows_v.at[b],
                              gsem.at[b]).wait()
        pltpu.async_copy(rows_v.at[b], acc.at[dst_v.at[j]], ssem.at[b],
                         add=True)
        return 0
    lax.fori_loop(0, NB, body, 0)

    pltpu.make_async_copy(rows_v.at[(NB - 1) % 2],
                          acc.at[dst_v.at[NB - 1]],
                          ssem.at[(NB - 1) % 2]).wait()

    plsc.subcore_barrier()

    pltpu.sync_copy(acc.at[pl.ds(sid * RPT, RPT)],
                    part.at[cid, pl.ds(sid * RPT, RPT)])


_FLAT = N * C
_FPW = _FLAT // NW
_CH = 10000


def _combine_body(scale_out, p_hbm, nin_hbm, h0a_hbm, nout_hbm, out_hbm,
                  p0_v, p1_v, nin_v, h0a_v, nout_v, o_v):
    wid = _worker_id()
    base = wid * _FPW

    def chunk(cnk, _):
        off = base + cnk * _CH
        pltpu.sync_copy(p_hbm.at[0, pl.ds(off, _CH)], p0_v)
        pltpu.sync_copy(p_hbm.at[1, pl.ds(off, _CH)], p1_v)
        pltpu.sync_copy(nin_hbm.at[pl.ds(off, _CH)], nin_v)
        pltpu.sync_copy(h0a_hbm.at[pl.ds(off, _CH)], h0a_v)
        if scale_out:
            pltpu.sync_copy(nout_hbm.at[pl.ds(off, _CH)], nout_v)

        def body(i, _):
            sl = pl.ds(i * 16, 16)
            v = (p0_v[sl] + p1_v[sl]) * nin_v[sl] + h0a_v[sl]
            if scale_out:
                v = v * nout_v[sl]
            o_v[sl] = v
            return 0
        lax.fori_loop(0, _CH // 16, body, 0)
        pltpu.sync_copy(o_v, out_hbm.at[pl.ds(off, _CH)])
        return 0
    lax.fori_loop(0, _FPW // _CH, chunk, 0)


def _make_combine(scale_out):
    return pl.kernel(
        functools.partial(_combine_body, scale_out),
        out_type=jax.ShapeDtypeStruct((_FLAT,), jnp.float32),
        mesh=_mesh,
        compiler_params=_sc_params,
        scratch_types=[
            pltpu.VMEM((_CH,), jnp.float32),
            pltpu.VMEM((_CH,), jnp.float32),
            pltpu.VMEM((_CH,), jnp.float32),
            pltpu.VMEM((_CH,), jnp.float32),
            pltpu.VMEM((_CH,), jnp.float32),
            pltpu.VMEM((_CH,), jnp.float32),
        ],
    )


_combine_mid = _make_combine(True)
_combine_last = _make_combine(False)


def kernel(features, edge_index, W0, b0, W1, b1, W2, b2):
    src = edge_index[0].reshape(NW, NB, CE)
    dst = edge_index[1].reshape(NW, NB, CE)

    dpart = _deg_kernel(src, dst)
    dpo = dpart[:, 0, :].reshape(NC, N, 1)
    dpi = dpart[:, 1, :].reshape(NC, N, 1)

    s0, h0a, nin1, nout = _mlp_kernel(features, W0, b0, W1, b1, W2, b2,
                                      dpo, dpi)

    h0a_f = h0a.reshape(_FLAT)
    nin1_f = nin1.reshape(_FLAT)
    nout_f = nout.reshape(_FLAT)

    s = s0
    for k in range(K_PROP):
        part = _step_kernel(s, src, dst)
        p_f = part.reshape(NC, _FLAT)
        if k < K_PROP - 1:
            s = _combine_mid(p_f, nin1_f, h0a_f, nout_f).reshape(N, C)
        else:
            s = _combine_last(p_f, nin1_f, h0a_f, nout_f).reshape(N, C)
    return s
